# baseline (speedup 1.0000x reference)
import functools

import jax
import jax.numpy as jnp
import numpy as np
from jax import lax
from jax.experimental import pallas as pl
from jax.experimental.pallas import tpu as pltpu
from jax.experimental.pallas import tpu_sc as plsc

NUM_FIELDS = 26
VOCAB = 100000
DENSE_DIM = 13
BATCH = 4096

NC = 2
NS = 16
L = 16
NW = NC * NS
B_PER_W = BATCH // NW
CHUNKS = B_PER_W // L
S_PER_W = B_PER_W * NUM_FIELDS
D_PER_W = B_PER_W * DENSE_DIM
N_GATH = S_PER_W // B_PER_W


def _sc_body(sparse_ref, dense_ref, table_ref, params_ref, offs_ref, out_ref,
             idx_v, gath_v, dense_v, w_v, off_v, out_v, sem):
    wid = lax.axis_index("s") * NC + lax.axis_index("c")

    pltpu.sync_copy(sparse_ref.at[pl.ds(wid * S_PER_W, S_PER_W)], idx_v)
    pltpu.sync_copy(offs_ref, off_v)
    pltpu.sync_copy(dense_ref.at[pl.ds(wid * D_PER_W, D_PER_W)], dense_v)
    pltpu.sync_copy(params_ref, w_v)

    def off_body(i, carry):
        sl = pl.ds(i * L, L)
        idx_v[sl] = idx_v[sl] + off_v[sl]
        return carry

    lax.fori_loop(0, S_PER_W // L, off_body, 0)

    copies = [
        pltpu.async_copy(
            table_ref.at[idx_v.at[pl.ds(k * B_PER_W, B_PER_W)]],
            gath_v.at[pl.ds(k * B_PER_W, B_PER_W)],
            sem,
        )
        for k in range(N_GATH)
    ]
    for cp in copies:
        cp.wait()

    wvec = w_v[:]
    lanes = lax.iota(jnp.int32, L)

    def sum_body(c, carry):
        row = c * L + lanes
        sidx = row * NUM_FIELDS
        acc = plsc.load_gather(gath_v, [sidx])
        for f in range(1, NUM_FIELDS):
            acc = acc + plsc.load_gather(gath_v, [sidx + f])
        didx = row * DENSE_DIM
        for d in range(DENSE_DIM):
            acc = acc + plsc.load_gather(dense_v, [didx + d]) * wvec[d]
        acc = acc + (wvec[DENSE_DIM] + wvec[DENSE_DIM + 1])
        out_v[pl.ds(c * L, L)] = 1.0 / (1.0 + jnp.exp(-acc))
        return carry

    lax.fori_loop(0, CHUNKS, sum_body, 0)

    pltpu.sync_copy(out_v, out_ref.at[pl.ds(wid * B_PER_W, B_PER_W)])


@jax.jit
def _run(sparse_flat, dense_flat, table_flat, params, offs):
    mesh = plsc.VectorSubcoreMesh(core_axis_name="c", subcore_axis_name="s")
    call = functools.partial(
        pl.kernel,
        mesh=mesh,
        compiler_params=pltpu.CompilerParams(needs_layout_passes=False),
        out_type=jax.ShapeDtypeStruct((BATCH,), jnp.float32),
        scratch_types=[
            pltpu.VMEM((S_PER_W,), jnp.int32),
            pltpu.VMEM((S_PER_W,), jnp.float32),
            pltpu.VMEM((D_PER_W,), jnp.float32),
            pltpu.VMEM((L,), jnp.float32),
            pltpu.VMEM((S_PER_W,), jnp.int32),
            pltpu.VMEM((B_PER_W,), jnp.float32),
            pltpu.SemaphoreType.DMA,
        ],
    )(_sc_body)
    return call(sparse_flat, dense_flat, table_flat, params, offs)


_OFFS = jnp.asarray(
    (np.arange(S_PER_W, dtype=np.int32) % NUM_FIELDS) * VOCAB, dtype=jnp.int32
)


def kernel(sparse_inputs, dense_inputs, tables, dense_W, dense_b, bias):
    sparse_flat = sparse_inputs.reshape(BATCH * NUM_FIELDS).astype(jnp.int32)
    dense_flat = dense_inputs.reshape(BATCH * DENSE_DIM)
    table_flat = tables.reshape(NUM_FIELDS * VOCAB)
    params = jnp.concatenate([
        dense_W.reshape(DENSE_DIM),
        dense_b.reshape(1),
        bias.reshape(1),
        jnp.zeros((1,), jnp.float32),
    ])
    return _run(sparse_flat, dense_flat, table_flat, params, _OFFS)


# speedup vs baseline: 2.7390x; 2.7390x over previous
import functools

import jax
import jax.numpy as jnp
from jax import lax
from jax.experimental import pallas as pl
from jax.experimental.pallas import tpu as pltpu
from jax.experimental.pallas import tpu_sc as plsc

NUM_FIELDS = 26
VOCAB = 100000
VOCAB_PAD = 100096
DENSE_DIM = 13
BATCH = 4096

NC = 2
NS = 16
L = 16
NW = NC * NS
B_PER_W = BATCH // NW
CHUNKS = B_PER_W // L


def _sc_body(sparse_ref, dense_ref, table_ref, params_ref, out_ref,
             idx_v, gath_v, dense_v, w_v, out_v, sem):
    wid = lax.axis_index("s") * NC + lax.axis_index("c")
    base = wid * B_PER_W

    pltpu.sync_copy(sparse_ref.at[:, pl.ds(base, B_PER_W)], idx_v)
    pltpu.sync_copy(dense_ref.at[:, pl.ds(base, B_PER_W)], dense_v)
    pltpu.sync_copy(params_ref, w_v)

    copies = [
        pltpu.async_copy(table_ref.at[f].at[idx_v.at[f]], gath_v.at[f], sem)
        for f in range(NUM_FIELDS)
    ]
    for cp in copies:
        cp.wait()

    wvec = w_v[:]

    def sum_body(c, carry):
        sl = pl.ds(c * L, L)
        acc = gath_v[0, sl]
        for f in range(1, NUM_FIELDS):
            acc = acc + gath_v[f, sl]
        for d in range(DENSE_DIM):
            acc = acc + dense_v[d, sl] * wvec[d]
        acc = acc + (wvec[DENSE_DIM] + wvec[DENSE_DIM + 1])
        out_v[sl] = 1.0 / (1.0 + jnp.exp(-acc))
        return carry

    lax.fori_loop(0, CHUNKS, sum_body, 0)

    pltpu.sync_copy(out_v, out_ref.at[pl.ds(base, B_PER_W)])


@jax.jit
def _run(sparse_t, dense_t, table2d, params):
    mesh = plsc.VectorSubcoreMesh(core_axis_name="c", subcore_axis_name="s")
    call = functools.partial(
        pl.kernel,
        mesh=mesh,
        compiler_params=pltpu.CompilerParams(
            needs_layout_passes=False, use_tc_tiling_on_sc=False
        ),
        out_type=jax.ShapeDtypeStruct((BATCH,), jnp.float32),
        scratch_types=[
            pltpu.VMEM((NUM_FIELDS, B_PER_W), jnp.int32),
            pltpu.VMEM((NUM_FIELDS, B_PER_W), jnp.float32),
            pltpu.VMEM((DENSE_DIM, B_PER_W), jnp.float32),
            pltpu.VMEM((L,), jnp.float32),
            pltpu.VMEM((B_PER_W,), jnp.float32),
            pltpu.SemaphoreType.DMA,
        ],
    )(_sc_body)
    return call(sparse_t, dense_t, table2d, params)


def kernel(sparse_inputs, dense_inputs, tables, dense_W, dense_b, bias):
    sparse_t = jnp.transpose(sparse_inputs).astype(jnp.int32)
    dense_t = jnp.transpose(dense_inputs)
    table2d = jnp.pad(
        tables, ((0, 0), (0, VOCAB_PAD - VOCAB), (0, 0))
    ).reshape(NUM_FIELDS, VOCAB_PAD)
    params = jnp.concatenate([
        dense_W.reshape(DENSE_DIM),
        dense_b.reshape(1),
        bias.reshape(1),
        jnp.zeros((1,), jnp.float32),
    ])
    return _run(sparse_t, dense_t, table2d, params)


# speedup vs baseline: 2.9885x; 1.0911x over previous
import functools

import jax
import jax.numpy as jnp
from jax import lax
from jax.experimental import pallas as pl
from jax.experimental.pallas import tpu as pltpu
from jax.experimental.pallas import tpu_sc as plsc

NUM_FIELDS = 26
VOCAB = 100000
DENSE_DIM = 13
BATCH = 4096

NC = 2
NS = 16
L = 16
NW = NC * NS
B_PER_W = BATCH // NW
CHUNKS = B_PER_W // L


def _sc_body(sparse_ref, dense_ref, table_ref, params_ref, out_ref,
             idx_v, gath_v, dense_v, w_v, out_v, sem):
    wid = lax.axis_index("s") * NC + lax.axis_index("c")
    base = wid * B_PER_W

    pltpu.sync_copy(sparse_ref.at[:, pl.ds(base, B_PER_W)], idx_v)
    pltpu.sync_copy(dense_ref.at[:, pl.ds(base, B_PER_W)], dense_v)
    pltpu.sync_copy(params_ref, w_v)

    copies = [
        pltpu.async_copy(table_ref.at[f].at[idx_v.at[f]], gath_v.at[f], sem)
        for f in range(NUM_FIELDS)
    ]
    for cp in copies:
        cp.wait()

    wvec = w_v[:]

    def sum_body(c, carry):
        sl = pl.ds(c * L, L)
        acc = gath_v[0, sl]
        for f in range(1, NUM_FIELDS):
            acc = acc + gath_v[f, sl]
        for d in range(DENSE_DIM):
            acc = acc + dense_v[d, sl] * wvec[d]
        acc = acc + (wvec[DENSE_DIM] + wvec[DENSE_DIM + 1])
        out_v[sl] = 1.0 / (1.0 + jnp.exp(-acc))
        return carry

    lax.fori_loop(0, CHUNKS, sum_body, 0)

    pltpu.sync_copy(out_v, out_ref.at[pl.ds(base, B_PER_W)])


@jax.jit
def _run(sparse_t, dense_t, table2d, params):
    mesh = plsc.VectorSubcoreMesh(core_axis_name="c", subcore_axis_name="s")
    call = functools.partial(
        pl.kernel,
        mesh=mesh,
        compiler_params=pltpu.CompilerParams(
            needs_layout_passes=False, use_tc_tiling_on_sc=False
        ),
        out_type=jax.ShapeDtypeStruct((BATCH,), jnp.float32),
        scratch_types=[
            pltpu.VMEM((NUM_FIELDS, B_PER_W), jnp.int32),
            pltpu.VMEM((NUM_FIELDS, B_PER_W), jnp.float32),
            pltpu.VMEM((DENSE_DIM, B_PER_W), jnp.float32),
            pltpu.VMEM((L,), jnp.float32),
            pltpu.VMEM((B_PER_W,), jnp.float32),
            pltpu.SemaphoreType.DMA,
        ],
    )(_sc_body)
    return call(sparse_t, dense_t, table2d, params)


def kernel(sparse_inputs, dense_inputs, tables, dense_W, dense_b, bias):
    sparse_t = jnp.transpose(sparse_inputs).astype(jnp.int32)
    dense_t = jnp.transpose(dense_inputs)
    table2d = lax.optimization_barrier(
        tables.reshape(NUM_FIELDS, VOCAB)
    )
    params = jnp.concatenate([
        dense_W.reshape(DENSE_DIM),
        dense_b.reshape(1),
        bias.reshape(1),
        jnp.zeros((1,), jnp.float32),
    ])
    return _run(sparse_t, dense_t, table2d, params)


# speedup vs baseline: 2.9987x; 1.0034x over previous
import functools

import jax
import jax.numpy as jnp
from jax import lax
from jax.experimental import pallas as pl
from jax.experimental.pallas import tpu as pltpu
from jax.experimental.pallas import tpu_sc as plsc

NUM_FIELDS = 26
VOCAB = 100000
DENSE_DIM = 13
BATCH = 4096

NC = 2
NS = 16
L = 16
NW = NC * NS
B_PER_W = BATCH // NW
CHUNKS = B_PER_W // L


def _sc_body(sparse_ref, dense_ref, table_ref, params_ref, out_ref,
             idx_v, gath_v, dense_v, w_v, out_v, sem):
    wid = lax.axis_index("s") * NC + lax.axis_index("c")
    base = wid * B_PER_W

    pltpu.sync_copy(sparse_ref.at[:, pl.ds(base, B_PER_W)], idx_v)

    copies = [
        pltpu.async_copy(table_ref.at[f].at[idx_v.at[f]], gath_v.at[f], sem)
        for f in range(NUM_FIELDS)
    ]

    pltpu.sync_copy(dense_ref.at[:, pl.ds(base, B_PER_W)], dense_v)
    pltpu.sync_copy(params_ref, w_v)
    wvec = w_v[:]

    slices = [pl.ds(c * L, L) for c in range(CHUNKS)]
    accs = []
    for c in range(CHUNKS):
        acc = dense_v[0, slices[c]] * wvec[0]
        for d in range(1, DENSE_DIM):
            acc = acc + dense_v[d, slices[c]] * wvec[d]
        accs.append(acc + (wvec[DENSE_DIM] + wvec[DENSE_DIM + 1]))

    for f in range(NUM_FIELDS):
        copies[f].wait()
        for c in range(CHUNKS):
            accs[c] = accs[c] + gath_v[f, slices[c]]

    for c in range(CHUNKS):
        out_v[slices[c]] = 1.0 / (1.0 + jnp.exp(-accs[c]))

    pltpu.sync_copy(out_v, out_ref.at[pl.ds(base, B_PER_W)])


@jax.jit
def _run(sparse_t, dense_t, table2d, params):
    mesh = plsc.VectorSubcoreMesh(core_axis_name="c", subcore_axis_name="s")
    call = functools.partial(
        pl.kernel,
        mesh=mesh,
        compiler_params=pltpu.CompilerParams(
            needs_layout_passes=False, use_tc_tiling_on_sc=False
        ),
        out_type=jax.ShapeDtypeStruct((BATCH,), jnp.float32),
        scratch_types=[
            pltpu.VMEM((NUM_FIELDS, B_PER_W), jnp.int32),
            pltpu.VMEM((NUM_FIELDS, B_PER_W), jnp.float32),
            pltpu.VMEM((DENSE_DIM, B_PER_W), jnp.float32),
            pltpu.VMEM((L,), jnp.float32),
            pltpu.VMEM((B_PER_W,), jnp.float32),
            pltpu.SemaphoreType.DMA,
        ],
    )(_sc_body)
    return call(sparse_t, dense_t, table2d, params)


def kernel(sparse_inputs, dense_inputs, tables, dense_W, dense_b, bias):
    sparse_t = jnp.transpose(sparse_inputs).astype(jnp.int32)
    dense_t = jnp.transpose(dense_inputs)
    table2d = lax.optimization_barrier(
        tables.reshape(NUM_FIELDS, VOCAB)
    )
    params = jnp.concatenate([
        dense_W.reshape(DENSE_DIM),
        dense_b.reshape(1),
        bias.reshape(1),
        jnp.zeros((1,), jnp.float32),
    ])
    return _run(sparse_t, dense_t, table2d, params)
